# baseline (speedup 1.0000x reference)
import functools

import jax
import jax.numpy as jnp
from jax import lax
from jax.experimental import pallas as pl
from jax.experimental.pallas import tpu as pltpu
from jax.experimental.pallas import tpu_sc as plsc

_HASH_SPECS = ((50, 32, 1.0), (200, 16, 0.5), (400, 8, 0.25), (400, 4, 0.125),
               (400, 3, 0.05))
_F_DIM = sum(d for _, d, _ in _HASH_SPECS)
_N = 131072
_NW = 32
_PTS_PER_W = _N // _NW
_CHUNK = 1024
_NCHUNK = _PTS_PER_W // _CHUNK
_GROUPS = _CHUNK // 16
_HA, _HB, _HC = 73856093, 19349663, 83492791

_TAB_STRIDE = [d + 1 if d % 2 == 0 else d for _, d, _ in _HASH_SPECS]
_TAB_OFF = []
_DIM_OFF = []
_t = _f = 0
for (_hs, _d, _), _st in zip(_HASH_SPECS, _TAB_STRIDE):
    _TAB_OFF.append(_t)
    _DIM_OFF.append(_f)
    _t += _hs * _st
    _f += _d
_TAB_TOTAL = _t


_MAGIC = {50: (0x51EB851F, 4), 200: (0x51EB851F, 6), 400: (0x51EB851F, 7)}


def _vmod(h, n):
    m, sh = _MAGIC[n]
    hu = h.astype(jnp.uint32)
    m1 = jnp.uint32(m >> 16)
    m0 = jnp.uint32(m & 0xFFFF)
    h1 = hu >> jnp.uint32(16)
    h0 = hu & jnp.uint32(0xFFFF)
    mid = h1 * m0 + h0 * m1 + ((h0 * m0) >> jnp.uint32(16))
    hi = h1 * m1 + (mid >> jnp.uint32(16))
    q = hi >> jnp.uint32(sh)
    return (hu - q * jnp.uint32(n)).astype(jnp.int32)


def _encode_group(tab_v, x0_v, x1_v, x2_v, f_v, g):
    gb = g * 16
    px = x0_v[pl.ds(gb, 16)]
    py = x1_v[pl.ds(gb, 16)]
    pz = x2_v[pl.ds(gb, 16)]
    for li, (hsize, dim, vs) in enumerate(_HASH_SPECS):
        vsf = jnp.float32(vs)
        if vs in (1.0, 0.5, 0.25, 0.125):
            inv = jnp.float32(1.0 / vs)
            p0 = px * inv
            p1 = py * inv
            p2 = pz * inv
        else:
            p0 = px / vsf
            p1 = py / vsf
            p2 = pz / vsf
        fl0 = p0.astype(jnp.int32)
        fl1 = p1.astype(jnp.int32)
        fl2 = p2.astype(jnp.int32)
        ff0 = fl0.astype(jnp.float32)
        ff1 = fl1.astype(jnp.float32)
        ff2 = fl2.astype(jnp.float32)
        ce0 = jnp.where(p0 != ff0, fl0 + 1, fl0)
        ce1 = jnp.where(p1 != ff1, fl1 + 1, fl1)
        ce2 = jnp.where(p2 != ff2, fl2 + 1, fl2)
        r0 = p0 - ff0
        r1 = p1 - ff1
        r2 = p2 - ff2
        wx = (r0, 1.0 - r0)
        wy = (r1, 1.0 - r1)
        wz = (r2, 1.0 - r2)
        ax = (fl0 * _HA, ce0 * _HA)
        ay = (fl1 * _HB, ce1 * _HB)
        az = (fl2 * _HC, ce2 * _HC)
        hb = []
        w = []
        for i in range(2):
            for j in range(2):
                hxy = ax[i] ^ ay[j]
                wxy = wx[i] * wy[j]
                for k in range(2):
                    h = _vmod(hxy ^ az[k], hsize)
                    hb.append(h * _TAB_STRIDE[li] + _TAB_OFF[li])
                    w.append(wxy * wz[k])
        doff = _DIM_OFF[li]
        for d in range(dim):
            t = [w[c] * plsc.load_gather(tab_v, [hb[c] + d if d else hb[c]])
                 for c in range(8)]
            acc = ((t[0] + t[1]) + (t[2] + t[3])) + ((t[4] + t[5]) + (t[6] + t[7]))
            f_v[doff + d, pl.ds(gb, 16)] = acc


@functools.partial(
    pl.kernel,
    out_type=jax.ShapeDtypeStruct((_F_DIM, _N), jnp.float32),
    mesh=plsc.VectorSubcoreMesh(core_axis_name="c", subcore_axis_name="s"),
    scratch_types=[
        pltpu.VMEM((_TAB_TOTAL,), jnp.float32),
        pltpu.VMEM((_CHUNK,), jnp.float32),
        pltpu.VMEM((_CHUNK,), jnp.float32),
        pltpu.VMEM((_CHUNK,), jnp.float32),
        pltpu.VMEM((_F_DIM, _CHUNK), jnp.float32),
    ],
    compiler_params=pltpu.CompilerParams(needs_layout_passes=False),
)
def _encode(x0_hbm, x1_hbm, x2_hbm, tab_hbm, out_hbm,
            tab_v, x0_v, x1_v, x2_v, f_v):
    wid = lax.axis_index("s") * 2 + lax.axis_index("c")
    pltpu.sync_copy(tab_hbm, tab_v)
    base = wid * _PTS_PER_W

    def chunk_body(ci, carry):
        cb = base + ci * _CHUNK
        pltpu.sync_copy(x0_hbm.at[pl.ds(cb, _CHUNK)], x0_v)
        pltpu.sync_copy(x1_hbm.at[pl.ds(cb, _CHUNK)], x1_v)
        pltpu.sync_copy(x2_hbm.at[pl.ds(cb, _CHUNK)], x2_v)

        @plsc.parallel_loop(0, _GROUPS)
        def group_body(g):
            _encode_group(tab_v, x0_v, x1_v, x2_v, f_v, g)
        pltpu.sync_copy(f_v, out_hbm.at[:, pl.ds(cb, _CHUNK)])
        return carry

    lax.fori_loop(0, _NCHUNK, chunk_body, 0)


_BN = 2048


def _mlp_body(ft_ref, dW1t_ref, db1_ref, dW2_ref, db2_ref,
              cW1t_ref, cb1_ref, cW2_ref, cb2_ref, dense_ref, color_ref):
    f = ft_ref[...]
    mm = (((1,), (0,)), ((), ()))
    tm = (((0,), (0,)), ((), ()))
    hp = dict(preferred_element_type=jnp.float32)
    hd = jnp.maximum(lax.dot_general(dW1t_ref[...], f, mm, **hp) + db1_ref[...],
                     0.0)
    dense_ref[...] = lax.dot_general(hd, dW2_ref[...], tm, **hp) + db2_ref[...]
    hc = jnp.maximum(lax.dot_general(cW1t_ref[...], f, mm, **hp) + cb1_ref[...],
                     0.0)
    color_ref[...] = lax.dot_general(hc, cW2_ref[...], tm, **hp) + cb2_ref[...]


def _mlp(ft, dW1, db1, dW2, db2, cW1, cb1, cW2, cb2):
    grid = (_N // _BN,)
    full = lambda shape: pl.BlockSpec(shape, lambda i: (0, 0))
    return pl.pallas_call(
        _mlp_body,
        grid=grid,
        in_specs=[
            pl.BlockSpec((_F_DIM, _BN), lambda i: (0, i)),
            full((_F_DIM, _F_DIM)), full((_F_DIM, 1)),
            full((_F_DIM, 1)), full((1, 1)),
            full((_F_DIM, _F_DIM)), full((_F_DIM, 1)),
            full((_F_DIM, 3)), full((1, 3)),
        ],
        out_specs=[
            pl.BlockSpec((_BN, 1), lambda i: (i, 0)),
            pl.BlockSpec((_BN, 3), lambda i: (i, 0)),
        ],
        out_shape=[
            jax.ShapeDtypeStruct((_N, 1), jnp.float32),
            jax.ShapeDtypeStruct((_N, 3), jnp.float32),
        ],
        compiler_params=pltpu.CompilerParams(
            dimension_semantics=("arbitrary",)),
    )(ft, dW1, db1, dW2, db2, cW1, cb1, cW2, cb2)


def kernel(x, emb0, emb1, emb2, emb3, emb4,
           dW1, db1, dW2, db2, cW1, cb1, cW2, cb2):
    parts = []
    for e, st in zip((emb0, emb1, emb2, emb3, emb4), _TAB_STRIDE):
        d = e.shape[1]
        if st != d:
            e = jnp.pad(e, ((0, 0), (0, st - d)))
        parts.append(e.reshape(-1))
    tab = jnp.concatenate(parts)
    x0 = x[:, 0]
    x1 = x[:, 1]
    x2 = x[:, 2]
    ft = _encode(x0, x1, x2, tab)
    dense, color = _mlp(ft, dW1.T, db1[:, None], dW2, db2[None, :],
                        cW1.T, cb1[:, None], cW2, cb2[None, :])
    return (dense, color)


# speedup vs baseline: 1.0517x; 1.0517x over previous
import functools

import jax
import jax.numpy as jnp
from jax import lax
from jax.experimental import pallas as pl
from jax.experimental.pallas import tpu as pltpu
from jax.experimental.pallas import tpu_sc as plsc

_HASH_SPECS = ((50, 32, 1.0), (200, 16, 0.5), (400, 8, 0.25), (400, 4, 0.125),
               (400, 3, 0.05))
_F_DIM = sum(d for _, d, _ in _HASH_SPECS)
_N = 131072
_NW = 32
_PTS_PER_W = _N // _NW
_CHUNK = 1024
_NCHUNK = _PTS_PER_W // _CHUNK
_GROUPS = _CHUNK // 16
_HA, _HB, _HC = 73856093, 19349663, 83492791

_TAB_STRIDE = [d + 1 if d % 2 == 0 else d for _, d, _ in _HASH_SPECS]
_TAB_OFF = []
_DIM_OFF = []
_t = _f = 0
for (_hs, _d, _), _st in zip(_HASH_SPECS, _TAB_STRIDE):
    _TAB_OFF.append(_t)
    _DIM_OFF.append(_f)
    _t += _hs * _st
    _f += _d
_TAB_TOTAL = _t


_MAGIC = {50: (0x51EB851F, 4), 200: (0x51EB851F, 6), 400: (0x51EB851F, 7)}


def _vmod(h, n):
    m, sh = _MAGIC[n]
    hu = h.astype(jnp.uint32)
    m1 = jnp.uint32(m >> 16)
    m0 = jnp.uint32(m & 0xFFFF)
    h1 = hu >> jnp.uint32(16)
    h0 = hu & jnp.uint32(0xFFFF)
    mid = h1 * m0 + h0 * m1 + ((h0 * m0) >> jnp.uint32(16))
    hi = h1 * m1 + (mid >> jnp.uint32(16))
    q = hi >> jnp.uint32(sh)
    return (hu - q * jnp.uint32(n)).astype(jnp.int32)


def _encode_group(tab_v, x0_v, x1_v, x2_v, f_v, g):
    gb = g * 16
    px = x0_v[pl.ds(gb, 16)]
    py = x1_v[pl.ds(gb, 16)]
    pz = x2_v[pl.ds(gb, 16)]
    for li, (hsize, dim, vs) in enumerate(_HASH_SPECS):
        vsf = jnp.float32(vs)
        if vs in (1.0, 0.5, 0.25, 0.125):
            inv = jnp.float32(1.0 / vs)
            p0 = px * inv
            p1 = py * inv
            p2 = pz * inv
        else:
            p0 = px / vsf
            p1 = py / vsf
            p2 = pz / vsf
        fl0 = p0.astype(jnp.int32)
        fl1 = p1.astype(jnp.int32)
        fl2 = p2.astype(jnp.int32)
        ff0 = fl0.astype(jnp.float32)
        ff1 = fl1.astype(jnp.float32)
        ff2 = fl2.astype(jnp.float32)
        ce0 = jnp.where(p0 != ff0, fl0 + 1, fl0)
        ce1 = jnp.where(p1 != ff1, fl1 + 1, fl1)
        ce2 = jnp.where(p2 != ff2, fl2 + 1, fl2)
        r0 = p0 - ff0
        r1 = p1 - ff1
        r2 = p2 - ff2
        wx = (r0, 1.0 - r0)
        wy = (r1, 1.0 - r1)
        wz = (r2, 1.0 - r2)
        ax = (fl0 * _HA, ce0 * _HA)
        ay = (fl1 * _HB, ce1 * _HB)
        az = (fl2 * _HC, ce2 * _HC)
        hb = []
        w = []
        for i in range(2):
            for j in range(2):
                hxy = ax[i] ^ ay[j]
                wxy = wx[i] * wy[j]
                for k in range(2):
                    h = _vmod(hxy ^ az[k], hsize)
                    hb.append(h * _TAB_STRIDE[li] + _TAB_OFF[li])
                    w.append(wxy * wz[k])
        doff = _DIM_OFF[li]
        for d in range(dim):
            t = [w[c] * plsc.load_gather(tab_v, [hb[c] + d if d else hb[c]])
                 for c in range(8)]
            acc = ((t[0] + t[1]) + (t[2] + t[3])) + ((t[4] + t[5]) + (t[6] + t[7]))
            f_v[doff + d, pl.ds(gb, 16)] = acc


@functools.partial(
    pl.kernel,
    out_type=jax.ShapeDtypeStruct((_F_DIM, _N), jnp.float32),
    mesh=plsc.VectorSubcoreMesh(core_axis_name="c", subcore_axis_name="s"),
    scratch_types=[
        pltpu.VMEM((_TAB_TOTAL,), jnp.float32),
        pltpu.VMEM((_CHUNK,), jnp.float32),
        pltpu.VMEM((_CHUNK,), jnp.float32),
        pltpu.VMEM((_CHUNK,), jnp.float32),
        pltpu.VMEM((_F_DIM, _CHUNK), jnp.float32),
    ],
    compiler_params=pltpu.CompilerParams(needs_layout_passes=False),
)
def _encode(x0_hbm, x1_hbm, x2_hbm, tab_hbm, out_hbm,
            tab_v, x0_v, x1_v, x2_v, f_v):
    wid = lax.axis_index("s") * 2 + lax.axis_index("c")
    pltpu.sync_copy(tab_hbm, tab_v)
    base = wid * _PTS_PER_W

    def chunk_body(ci, carry):
        cb = base + ci * _CHUNK
        pltpu.sync_copy(x0_hbm.at[pl.ds(cb, _CHUNK)], x0_v)
        pltpu.sync_copy(x1_hbm.at[pl.ds(cb, _CHUNK)], x1_v)
        pltpu.sync_copy(x2_hbm.at[pl.ds(cb, _CHUNK)], x2_v)

        @plsc.parallel_loop(0, _GROUPS)
        def group_body(g):
            _encode_group(tab_v, x0_v, x1_v, x2_v, f_v, g)
        pltpu.sync_copy(f_v, out_hbm.at[:, pl.ds(cb, _CHUNK)])
        return carry

    lax.fori_loop(0, _NCHUNK, chunk_body, 0)


_BN = 4096


def _mlp_body(ft_ref, dW1t_ref, db1_ref, dW2_ref, db2_ref,
              cW1t_ref, cb1_ref, cW2_ref, cb2_ref, dense_ref, color_ref):
    f = ft_ref[...]
    mm = (((1,), (0,)), ((), ()))
    tm = (((0,), (0,)), ((), ()))
    hp = dict(preferred_element_type=jnp.float32)
    hd = jnp.maximum(lax.dot_general(dW1t_ref[...], f, mm, **hp) + db1_ref[...],
                     0.0)
    dense_ref[...] = lax.dot_general(hd, dW2_ref[...], tm, **hp) + db2_ref[...]
    hc = jnp.maximum(lax.dot_general(cW1t_ref[...], f, mm, **hp) + cb1_ref[...],
                     0.0)
    color_ref[...] = lax.dot_general(hc, cW2_ref[...], tm, **hp) + cb2_ref[...]


def _mlp(ft, dW1, db1, dW2, db2, cW1, cb1, cW2, cb2):
    grid = (_N // _BN,)
    full = lambda shape: pl.BlockSpec(shape, lambda i: (0, 0))
    return pl.pallas_call(
        _mlp_body,
        grid=grid,
        in_specs=[
            pl.BlockSpec((_F_DIM, _BN), lambda i: (0, i)),
            full((_F_DIM, _F_DIM)), full((_F_DIM, 1)),
            full((_F_DIM, 1)), full((1, 1)),
            full((_F_DIM, _F_DIM)), full((_F_DIM, 1)),
            full((_F_DIM, 3)), full((1, 3)),
        ],
        out_specs=[
            pl.BlockSpec((_BN, 1), lambda i: (i, 0)),
            pl.BlockSpec((_BN, 3), lambda i: (i, 0)),
        ],
        out_shape=[
            jax.ShapeDtypeStruct((_N, 1), jnp.float32),
            jax.ShapeDtypeStruct((_N, 3), jnp.float32),
        ],
        compiler_params=pltpu.CompilerParams(
            dimension_semantics=("arbitrary",)),
    )(ft, dW1, db1, dW2, db2, cW1, cb1, cW2, cb2)


def kernel(x, emb0, emb1, emb2, emb3, emb4,
           dW1, db1, dW2, db2, cW1, cb1, cW2, cb2):
    parts = []
    for e, st in zip((emb0, emb1, emb2, emb3, emb4), _TAB_STRIDE):
        d = e.shape[1]
        if st != d:
            e = jnp.pad(e, ((0, 0), (0, st - d)))
        parts.append(e.reshape(-1))
    tab = jnp.concatenate(parts)
    x0 = x[:, 0]
    x1 = x[:, 1]
    x2 = x[:, 2]
    ft = _encode(x0, x1, x2, tab)
    dense, color = _mlp(ft, dW1.T, db1[:, None], dW2, db2[None, :],
                        cW1.T, cb1[:, None], cW2, cb2[None, :])
    return (dense, color)


# speedup vs baseline: 1.3410x; 1.2751x over previous
import functools

import jax
import jax.numpy as jnp
from jax import lax
from jax.experimental import pallas as pl
from jax.experimental.pallas import tpu as pltpu
from jax.experimental.pallas import tpu_sc as plsc

_HASH_SPECS = ((50, 32, 1.0), (200, 16, 0.5), (400, 8, 0.25), (400, 4, 0.125),
               (400, 3, 0.05))
_F_DIM = sum(d for _, d, _ in _HASH_SPECS)
_N = 131072
_NW = 32
_PTS_PER_W = _N // _NW
_CHUNK = 1024
_NCHUNK = _PTS_PER_W // _CHUNK
_GROUPS = _CHUNK // 16
_HA, _HB, _HC = 73856093, 19349663, 83492791

_TAB_STRIDE = [d + 1 if d % 2 == 0 else d for _, d, _ in _HASH_SPECS]
_TAB_OFF = []
_DIM_OFF = []
_t = _f = 0
for (_hs, _d, _), _st in zip(_HASH_SPECS, _TAB_STRIDE):
    _TAB_OFF.append(_t)
    _DIM_OFF.append(_f)
    _t += _hs * _st
    _f += _d
_TAB_TOTAL = _t


_MAGIC = {50: (0x51EB851F, 4), 200: (0x51EB851F, 6), 400: (0x51EB851F, 7)}


def _vmod(h, n):
    m, sh = _MAGIC[n]
    hu = h.astype(jnp.uint32)
    m1 = jnp.uint32(m >> 16)
    m0 = jnp.uint32(m & 0xFFFF)
    h1 = hu >> jnp.uint32(16)
    h0 = hu & jnp.uint32(0xFFFF)
    mid = h1 * m0 + h0 * m1 + ((h0 * m0) >> jnp.uint32(16))
    hi = h1 * m1 + (mid >> jnp.uint32(16))
    q = hi >> jnp.uint32(sh)
    return (hu - q * jnp.uint32(n)).astype(jnp.int32)


def _encode_group(tab_v, x0_v, x1_v, x2_v, f_v, g):
    gb = g * 16
    px = x0_v[pl.ds(gb, 16)]
    py = x1_v[pl.ds(gb, 16)]
    pz = x2_v[pl.ds(gb, 16)]
    for li, (hsize, dim, vs) in enumerate(_HASH_SPECS):
        vsf = jnp.float32(vs)
        if vs in (1.0, 0.5, 0.25, 0.125):
            inv = jnp.float32(1.0 / vs)
            p0 = px * inv
            p1 = py * inv
            p2 = pz * inv
        else:
            p0 = px / vsf
            p1 = py / vsf
            p2 = pz / vsf
        fl0 = p0.astype(jnp.int32)
        fl1 = p1.astype(jnp.int32)
        fl2 = p2.astype(jnp.int32)
        ff0 = fl0.astype(jnp.float32)
        ff1 = fl1.astype(jnp.float32)
        ff2 = fl2.astype(jnp.float32)
        ce0 = jnp.where(p0 != ff0, fl0 + 1, fl0)
        ce1 = jnp.where(p1 != ff1, fl1 + 1, fl1)
        ce2 = jnp.where(p2 != ff2, fl2 + 1, fl2)
        r0 = p0 - ff0
        r1 = p1 - ff1
        r2 = p2 - ff2
        wx = (r0, 1.0 - r0)
        wy = (r1, 1.0 - r1)
        wz = (r2, 1.0 - r2)
        ax = (fl0 * _HA, ce0 * _HA)
        ay = (fl1 * _HB, ce1 * _HB)
        az = (fl2 * _HC, ce2 * _HC)
        hb = []
        w = []
        for i in range(2):
            for j in range(2):
                hxy = ax[i] ^ ay[j]
                wxy = wx[i] * wy[j]
                for k in range(2):
                    h = _vmod(hxy ^ az[k], hsize)
                    hb.append(h * _TAB_STRIDE[li] + _TAB_OFF[li])
                    w.append(wxy * wz[k])
        doff = _DIM_OFF[li]
        for d in range(dim):
            t = [w[c] * plsc.load_gather(tab_v, [hb[c] + d if d else hb[c]])
                 for c in range(8)]
            acc = ((t[0] + t[1]) + (t[2] + t[3])) + ((t[4] + t[5]) + (t[6] + t[7]))
            f_v[doff + d, pl.ds(gb, 16)] = acc


@functools.partial(
    pl.kernel,
    out_type=jax.ShapeDtypeStruct((_F_DIM, _N), jnp.float32),
    mesh=plsc.VectorSubcoreMesh(core_axis_name="c", subcore_axis_name="s"),
    scratch_types=[
        pltpu.VMEM((_TAB_TOTAL,), jnp.float32),
        pltpu.VMEM((_CHUNK,), jnp.float32),
        pltpu.VMEM((_CHUNK,), jnp.float32),
        pltpu.VMEM((_CHUNK,), jnp.float32),
        pltpu.VMEM((_F_DIM, _CHUNK), jnp.float32),
    ],
    compiler_params=pltpu.CompilerParams(needs_layout_passes=False),
)
def _encode(x0_hbm, x1_hbm, x2_hbm, tab_hbm, out_hbm,
            tab_v, x0_v, x1_v, x2_v, f_v):
    wid = lax.axis_index("s") * 2 + lax.axis_index("c")
    pltpu.sync_copy(tab_hbm, tab_v)
    base = wid * _PTS_PER_W

    def chunk_body(ci, carry):
        cb = base + ci * _CHUNK
        pltpu.sync_copy(x0_hbm.at[pl.ds(cb, _CHUNK)], x0_v)
        pltpu.sync_copy(x1_hbm.at[pl.ds(cb, _CHUNK)], x1_v)
        pltpu.sync_copy(x2_hbm.at[pl.ds(cb, _CHUNK)], x2_v)

        @plsc.parallel_loop(0, _GROUPS)
        def group_body(g):
            _encode_group(tab_v, x0_v, x1_v, x2_v, f_v, g)
        pltpu.sync_copy(f_v, out_hbm.at[:, pl.ds(cb, _CHUNK)])
        return carry

    lax.fori_loop(0, _NCHUNK, chunk_body, 0)


_BN = 4096


def _mlp_body(ft_ref, dW1t_ref, db1_ref, dW2_ref, db2_ref,
              cW1t_ref, cb1_ref, cW2_ref, cb2_ref, dense_ref, color_ref):
    f = ft_ref[...]
    mm = (((1,), (0,)), ((), ()))
    hp = dict(preferred_element_type=jnp.float32)
    hd = jnp.maximum(lax.dot_general(dW1t_ref[...], f, mm, **hp) + db1_ref[...],
                     0.0)
    dense_ref[...] = (jnp.sum(hd * dW2_ref[...], axis=0, keepdims=True)
                      + db2_ref[...])
    hc = jnp.maximum(lax.dot_general(cW1t_ref[...], f, mm, **hp) + cb1_ref[...],
                     0.0)
    color_ref[...] = jnp.concatenate(
        [jnp.sum(hc * cW2_ref[...][:, c:c + 1], axis=0, keepdims=True)
         for c in range(3)], axis=0) + cb2_ref[...]


def _mlp(ft, dW1, db1, dW2, db2, cW1, cb1, cW2, cb2):
    grid = (_N // _BN,)
    full = lambda shape: pl.BlockSpec(shape, lambda i: (0, 0))
    return pl.pallas_call(
        _mlp_body,
        grid=grid,
        in_specs=[
            pl.BlockSpec((_F_DIM, _BN), lambda i: (0, i)),
            full((_F_DIM, _F_DIM)), full((_F_DIM, 1)),
            full((_F_DIM, 1)), full((1, 1)),
            full((_F_DIM, _F_DIM)), full((_F_DIM, 1)),
            full((_F_DIM, 3)), full((3, 1)),
        ],
        out_specs=[
            pl.BlockSpec((1, _BN), lambda i: (0, i)),
            pl.BlockSpec((3, _BN), lambda i: (0, i)),
        ],
        out_shape=[
            jax.ShapeDtypeStruct((1, _N), jnp.float32),
            jax.ShapeDtypeStruct((3, _N), jnp.float32),
        ],
        compiler_params=pltpu.CompilerParams(
            dimension_semantics=("arbitrary",)),
    )(ft, dW1, db1, dW2, db2, cW1, cb1, cW2, cb2)


def kernel(x, emb0, emb1, emb2, emb3, emb4,
           dW1, db1, dW2, db2, cW1, cb1, cW2, cb2):
    parts = []
    for e, st in zip((emb0, emb1, emb2, emb3, emb4), _TAB_STRIDE):
        d = e.shape[1]
        if st != d:
            e = jnp.pad(e, ((0, 0), (0, st - d)))
        parts.append(e.reshape(-1))
    tab = jnp.concatenate(parts)
    x0 = x[:, 0]
    x1 = x[:, 1]
    x2 = x[:, 2]
    ft = _encode(x0, x1, x2, tab)
    dense_r, color_r = _mlp(ft, dW1.T, db1[:, None], dW2, db2[:, None],
                            cW1.T, cb1[:, None], cW2, cb2[:, None])
    return (dense_r.reshape(-1, 1), color_r.T)


# speedup vs baseline: 1.4758x; 1.1005x over previous
import functools

import jax
import jax.numpy as jnp
from jax import lax
from jax.experimental import pallas as pl
from jax.experimental.pallas import tpu as pltpu
from jax.experimental.pallas import tpu_sc as plsc

_HASH_SPECS = ((50, 32, 1.0), (200, 16, 0.5), (400, 8, 0.25), (400, 4, 0.125),
               (400, 3, 0.05))
_F_DIM = sum(d for _, d, _ in _HASH_SPECS)
_N = 131072
_NW = 32
_PTS_PER_W = _N // _NW
_CHUNK = 1024
_NCHUNK = _PTS_PER_W // _CHUNK
_GROUPS = _CHUNK // 16
_HA, _HB, _HC = 73856093, 19349663, 83492791

_COL_STRIDE = [(hs + 7) // 8 * 8 for hs, _, _ in _HASH_SPECS]
_COL_OFF = []
_DIM_OFF = []
_t = _f = 0
for (_hs, _d, _), _cs in zip(_HASH_SPECS, _COL_STRIDE):
    _COL_OFF.append([_t + _cs * _i for _i in range(_d)])
    _DIM_OFF.append(_f)
    _t += _cs * _d
    _f += _d
_TAB_TOTAL = _t


_MAGIC = {50: (0x51EB851F, 4), 200: (0x51EB851F, 6), 400: (0x51EB851F, 7)}


def _vmod(h, n):
    m, sh = _MAGIC[n]
    hu = h.astype(jnp.uint32)
    m1 = jnp.uint32(m >> 16)
    m0 = jnp.uint32(m & 0xFFFF)
    h1 = hu >> jnp.uint32(16)
    h0 = hu & jnp.uint32(0xFFFF)
    mid = h1 * m0 + h0 * m1 + ((h0 * m0) >> jnp.uint32(16))
    hi = h1 * m1 + (mid >> jnp.uint32(16))
    q = hi >> jnp.uint32(sh)
    return (hu - q * jnp.uint32(n)).astype(jnp.int32)


def _encode_group(tab_v, x0_v, x1_v, x2_v, f_v, g):
    gb = g * 16
    px = x0_v[pl.ds(gb, 16)]
    py = x1_v[pl.ds(gb, 16)]
    pz = x2_v[pl.ds(gb, 16)]
    for li, (hsize, dim, vs) in enumerate(_HASH_SPECS):
        vsf = jnp.float32(vs)
        if vs in (1.0, 0.5, 0.25, 0.125):
            inv = jnp.float32(1.0 / vs)
            p0 = px * inv
            p1 = py * inv
            p2 = pz * inv
        else:
            p0 = px / vsf
            p1 = py / vsf
            p2 = pz / vsf
        fl0 = p0.astype(jnp.int32)
        fl1 = p1.astype(jnp.int32)
        fl2 = p2.astype(jnp.int32)
        ff0 = fl0.astype(jnp.float32)
        ff1 = fl1.astype(jnp.float32)
        ff2 = fl2.astype(jnp.float32)
        ce0 = jnp.where(p0 != ff0, fl0 + 1, fl0)
        ce1 = jnp.where(p1 != ff1, fl1 + 1, fl1)
        ce2 = jnp.where(p2 != ff2, fl2 + 1, fl2)
        r0 = p0 - ff0
        r1 = p1 - ff1
        r2 = p2 - ff2
        wx = (r0, 1.0 - r0)
        wy = (r1, 1.0 - r1)
        wz = (r2, 1.0 - r2)
        ax = (fl0 * _HA, ce0 * _HA)
        ay = (fl1 * _HB, ce1 * _HB)
        az = (fl2 * _HC, ce2 * _HC)
        hb = []
        w = []
        for i in range(2):
            for j in range(2):
                hxy = ax[i] ^ ay[j]
                wxy = wx[i] * wy[j]
                for k in range(2):
                    hb.append(_vmod(hxy ^ az[k], hsize))
                    w.append(wxy * wz[k])
        doff = _DIM_OFF[li]
        for d in range(dim):
            tv = tab_v.at[pl.ds(_COL_OFF[li][d], hsize)]
            t = [w[c] * plsc.load_gather(tv, [hb[c]]) for c in range(8)]
            acc = ((t[0] + t[1]) + (t[2] + t[3])) + ((t[4] + t[5]) + (t[6] + t[7]))
            f_v[doff + d, pl.ds(gb, 16)] = acc


@functools.partial(
    pl.kernel,
    out_type=jax.ShapeDtypeStruct((_F_DIM, _N), jnp.float32),
    mesh=plsc.VectorSubcoreMesh(core_axis_name="c", subcore_axis_name="s"),
    scratch_types=[
        pltpu.VMEM((_TAB_TOTAL,), jnp.float32),
        pltpu.VMEM((_CHUNK,), jnp.float32),
        pltpu.VMEM((_CHUNK,), jnp.float32),
        pltpu.VMEM((_CHUNK,), jnp.float32),
        pltpu.VMEM((_F_DIM, _CHUNK), jnp.float32),
    ],
    compiler_params=pltpu.CompilerParams(needs_layout_passes=False),
)
def _encode(x0_hbm, x1_hbm, x2_hbm, tab_hbm, out_hbm,
            tab_v, x0_v, x1_v, x2_v, f_v):
    wid = lax.axis_index("s") * 2 + lax.axis_index("c")
    pltpu.sync_copy(tab_hbm, tab_v)
    base = wid * _PTS_PER_W

    def chunk_body(ci, carry):
        cb = base + ci * _CHUNK
        pltpu.sync_copy(x0_hbm.at[pl.ds(cb, _CHUNK)], x0_v)
        pltpu.sync_copy(x1_hbm.at[pl.ds(cb, _CHUNK)], x1_v)
        pltpu.sync_copy(x2_hbm.at[pl.ds(cb, _CHUNK)], x2_v)

        @plsc.parallel_loop(0, _GROUPS)
        def group_body(g):
            _encode_group(tab_v, x0_v, x1_v, x2_v, f_v, g)
        pltpu.sync_copy(f_v, out_hbm.at[:, pl.ds(cb, _CHUNK)])
        return carry

    lax.fori_loop(0, _NCHUNK, chunk_body, 0)


_BN = 4096


def _mlp_body(ft_ref, dW1t_ref, db1_ref, dW2_ref, db2_ref,
              cW1t_ref, cb1_ref, cW2_ref, cb2_ref, dense_ref, color_ref):
    f = ft_ref[...]
    mm = (((1,), (0,)), ((), ()))
    hp = dict(preferred_element_type=jnp.float32)
    hd = jnp.maximum(lax.dot_general(dW1t_ref[...], f, mm, **hp) + db1_ref[...],
                     0.0)
    dense_ref[...] = (jnp.sum(hd * dW2_ref[...], axis=0, keepdims=True)
                      + db2_ref[...])
    hc = jnp.maximum(lax.dot_general(cW1t_ref[...], f, mm, **hp) + cb1_ref[...],
                     0.0)
    color_ref[...] = jnp.concatenate(
        [jnp.sum(hc * cW2_ref[...][:, c:c + 1], axis=0, keepdims=True)
         for c in range(3)], axis=0) + cb2_ref[...]


def _mlp(ft, dW1, db1, dW2, db2, cW1, cb1, cW2, cb2):
    grid = (_N // _BN,)
    full = lambda shape: pl.BlockSpec(shape, lambda i: (0, 0))
    return pl.pallas_call(
        _mlp_body,
        grid=grid,
        in_specs=[
            pl.BlockSpec((_F_DIM, _BN), lambda i: (0, i)),
            full((_F_DIM, _F_DIM)), full((_F_DIM, 1)),
            full((_F_DIM, 1)), full((1, 1)),
            full((_F_DIM, _F_DIM)), full((_F_DIM, 1)),
            full((_F_DIM, 3)), full((3, 1)),
        ],
        out_specs=[
            pl.BlockSpec((1, _BN), lambda i: (0, i)),
            pl.BlockSpec((3, _BN), lambda i: (0, i)),
        ],
        out_shape=[
            jax.ShapeDtypeStruct((1, _N), jnp.float32),
            jax.ShapeDtypeStruct((3, _N), jnp.float32),
        ],
        compiler_params=pltpu.CompilerParams(
            dimension_semantics=("arbitrary",)),
    )(ft, dW1, db1, dW2, db2, cW1, cb1, cW2, cb2)


def kernel(x, emb0, emb1, emb2, emb3, emb4,
           dW1, db1, dW2, db2, cW1, cb1, cW2, cb2):
    parts = []
    for e, cs in zip((emb0, emb1, emb2, emb3, emb4), _COL_STRIDE):
        et = e.T
        hs = et.shape[1]
        if cs != hs:
            et = jnp.pad(et, ((0, 0), (0, cs - hs)))
        parts.append(et.reshape(-1))
    tab = jnp.concatenate(parts)
    x0 = x[:, 0]
    x1 = x[:, 1]
    x2 = x[:, 2]
    ft = _encode(x0, x1, x2, tab)
    dense_r, color_r = _mlp(ft, dW1.T, db1[:, None], dW2, db2[:, None],
                            cW1.T, cb1[:, None], cW2, cb2[:, None])
    return (dense_r.reshape(-1, 1), color_r.T)


# speedup vs baseline: 1.5385x; 1.0425x over previous
import functools

import jax
import jax.numpy as jnp
from jax import lax
from jax.experimental import pallas as pl
from jax.experimental.pallas import tpu as pltpu
from jax.experimental.pallas import tpu_sc as plsc

_HASH_SPECS = ((50, 32, 1.0), (200, 16, 0.5), (400, 8, 0.25), (400, 4, 0.125),
               (400, 3, 0.05))
_F_DIM = sum(d for _, d, _ in _HASH_SPECS)
_N = 131072
_NW = 32
_PTS_PER_W = _N // _NW
_CHUNK = 1024
_NCHUNK = _PTS_PER_W // _CHUNK
_GROUPS = _CHUNK // 16
_HA, _HB, _HC = 73856093, 19349663, 83492791

_COL_STRIDE = [(hs + 7) // 8 * 8 for hs, _, _ in _HASH_SPECS]
_COL_OFF = []
_DIM_OFF = []
_t = _f = 0
for (_hs, _d, _), _cs in zip(_HASH_SPECS, _COL_STRIDE):
    _COL_OFF.append([_t + _cs * _i for _i in range(_d)])
    _DIM_OFF.append(_f)
    _t += _cs * _d
    _f += _d
_TAB_TOTAL = _t

_GRID_S = [2, 3, 5, 9, 21]
_LUT_SZP = [8, 8, 8, 16, 24]
_LUT_OFF = []
_l = 0
for _s, _zp in zip(_GRID_S, _LUT_SZP):
    _LUT_OFF.append(_l)
    _l += _s * _s * _zp
_LUT_TOTAL = _l


_MAGIC = {50: (0x51EB851F, 4), 200: (0x51EB851F, 6), 400: (0x51EB851F, 7)}


def _vmod(h, n):
    m, sh = _MAGIC[n]
    hu = h.astype(jnp.uint32)
    m1 = jnp.uint32(m >> 16)
    m0 = jnp.uint32(m & 0xFFFF)
    h1 = hu >> jnp.uint32(16)
    h0 = hu & jnp.uint32(0xFFFF)
    mid = h1 * m0 + h0 * m1 + ((h0 * m0) >> jnp.uint32(16))
    hi = h1 * m1 + (mid >> jnp.uint32(16))
    q = hi >> jnp.uint32(sh)
    return (hu - q * jnp.uint32(n)).astype(jnp.int32)


def _build_luts(lut_v):
    for li, (hsize, dim, vs) in enumerate(_HASH_SPECS):
        s = _GRID_S[li]
        szp = _LUT_SZP[li]
        base = _LUT_OFF[li]
        zoffs = [0] if s <= 16 else [0, szp - 16]
        cz16 = jnp.arange(16, dtype=jnp.int32)

        def cx_body(cx, carry, s=s, szp=szp, base=base, zoffs=zoffs,
                    cz16=cz16, hsize=hsize):
            def cy_body(cy, c2):
                hxy = (cx * _HA) ^ (cy * _HB)
                region = base + (cx * s + cy) * szp
                for zo in zoffs:
                    h = _vmod(hxy ^ ((cz16 + zo) * _HC), hsize)
                    lut_v[pl.ds(region + zo, 16)] = h
                return c2
            lax.fori_loop(0, s, cy_body, 0)
            return carry

        lax.fori_loop(0, s, cx_body, 0)


def _encode_group(tab_v, lut_v, x0_v, x1_v, x2_v, f_v, g):
    gb = g * 16
    px = x0_v[pl.ds(gb, 16)]
    py = x1_v[pl.ds(gb, 16)]
    pz = x2_v[pl.ds(gb, 16)]
    for li, (hsize, dim, vs) in enumerate(_HASH_SPECS):
        vsf = jnp.float32(vs)
        if vs in (1.0, 0.5, 0.25, 0.125):
            inv = jnp.float32(1.0 / vs)
            p0 = px * inv
            p1 = py * inv
            p2 = pz * inv
        else:
            p0 = px / vsf
            p1 = py / vsf
            p2 = pz / vsf
        fl0 = p0.astype(jnp.int32)
        fl1 = p1.astype(jnp.int32)
        fl2 = p2.astype(jnp.int32)
        ff0 = fl0.astype(jnp.float32)
        ff1 = fl1.astype(jnp.float32)
        ff2 = fl2.astype(jnp.float32)
        ce0 = jnp.where(p0 != ff0, fl0 + 1, fl0)
        ce1 = jnp.where(p1 != ff1, fl1 + 1, fl1)
        ce2 = jnp.where(p2 != ff2, fl2 + 1, fl2)
        r0 = p0 - ff0
        r1 = p1 - ff1
        r2 = p2 - ff2
        wx = (r0, 1.0 - r0)
        wy = (r1, 1.0 - r1)
        wz = (r2, 1.0 - r2)
        s = _GRID_S[li]
        szp = _LUT_SZP[li]
        lt = lut_v.at[pl.ds(_LUT_OFF[li], s * s * szp)]
        ax = (fl0 * (s * szp), ce0 * (s * szp))
        ay = (fl1 * szp, ce1 * szp)
        az = (fl2, ce2)
        hb = []
        w = []
        for i in range(2):
            for j in range(2):
                axy = ax[i] + ay[j]
                wxy = wx[i] * wy[j]
                for k in range(2):
                    hb.append(plsc.load_gather(lt, [axy + az[k]]))
                    w.append(wxy * wz[k])
        doff = _DIM_OFF[li]
        for d in range(dim):
            tv = tab_v.at[pl.ds(_COL_OFF[li][d], hsize)]
            t = [w[c] * plsc.load_gather(tv, [hb[c]]) for c in range(8)]
            acc = ((t[0] + t[1]) + (t[2] + t[3])) + ((t[4] + t[5]) + (t[6] + t[7]))
            f_v[doff + d, pl.ds(gb, 16)] = acc


@functools.partial(
    pl.kernel,
    out_type=jax.ShapeDtypeStruct((_F_DIM, _N), jnp.float32),
    mesh=plsc.VectorSubcoreMesh(core_axis_name="c", subcore_axis_name="s"),
    scratch_types=[
        pltpu.VMEM((_TAB_TOTAL,), jnp.float32),
        pltpu.VMEM((_LUT_TOTAL,), jnp.int32),
        pltpu.VMEM((_CHUNK,), jnp.float32),
        pltpu.VMEM((_CHUNK,), jnp.float32),
        pltpu.VMEM((_CHUNK,), jnp.float32),
        pltpu.VMEM((_F_DIM, _CHUNK), jnp.float32),
    ],
    compiler_params=pltpu.CompilerParams(needs_layout_passes=False),
)
def _encode(x0_hbm, x1_hbm, x2_hbm, tab_hbm, out_hbm,
            tab_v, lut_v, x0_v, x1_v, x2_v, f_v):
    wid = lax.axis_index("s") * 2 + lax.axis_index("c")
    pltpu.sync_copy(tab_hbm, tab_v)
    _build_luts(lut_v)
    base = wid * _PTS_PER_W

    def chunk_body(ci, carry):
        cb = base + ci * _CHUNK
        pltpu.sync_copy(x0_hbm.at[pl.ds(cb, _CHUNK)], x0_v)
        pltpu.sync_copy(x1_hbm.at[pl.ds(cb, _CHUNK)], x1_v)
        pltpu.sync_copy(x2_hbm.at[pl.ds(cb, _CHUNK)], x2_v)

        @plsc.parallel_loop(0, _GROUPS)
        def group_body(g):
            _encode_group(tab_v, lut_v, x0_v, x1_v, x2_v, f_v, g)
        pltpu.sync_copy(f_v, out_hbm.at[:, pl.ds(cb, _CHUNK)])
        return carry

    lax.fori_loop(0, _NCHUNK, chunk_body, 0)


_BN = 4096


def _mlp_body(ft_ref, dW1t_ref, db1_ref, dW2_ref, db2_ref,
              cW1t_ref, cb1_ref, cW2_ref, cb2_ref, dense_ref, color_ref):
    f = ft_ref[...]
    mm = (((1,), (0,)), ((), ()))
    hp = dict(preferred_element_type=jnp.float32)
    hd = jnp.maximum(lax.dot_general(dW1t_ref[...], f, mm, **hp) + db1_ref[...],
                     0.0)
    dense_ref[...] = (jnp.sum(hd * dW2_ref[...], axis=0, keepdims=True)
                      + db2_ref[...])
    hc = jnp.maximum(lax.dot_general(cW1t_ref[...], f, mm, **hp) + cb1_ref[...],
                     0.0)
    color_ref[...] = jnp.concatenate(
        [jnp.sum(hc * cW2_ref[...][:, c:c + 1], axis=0, keepdims=True)
         for c in range(3)], axis=0) + cb2_ref[...]


def _mlp(ft, dW1, db1, dW2, db2, cW1, cb1, cW2, cb2):
    grid = (_N // _BN,)
    full = lambda shape: pl.BlockSpec(shape, lambda i: (0, 0))
    return pl.pallas_call(
        _mlp_body,
        grid=grid,
        in_specs=[
            pl.BlockSpec((_F_DIM, _BN), lambda i: (0, i)),
            full((_F_DIM, _F_DIM)), full((_F_DIM, 1)),
            full((_F_DIM, 1)), full((1, 1)),
            full((_F_DIM, _F_DIM)), full((_F_DIM, 1)),
            full((_F_DIM, 3)), full((3, 1)),
        ],
        out_specs=[
            pl.BlockSpec((1, _BN), lambda i: (0, i)),
            pl.BlockSpec((3, _BN), lambda i: (0, i)),
        ],
        out_shape=[
            jax.ShapeDtypeStruct((1, _N), jnp.float32),
            jax.ShapeDtypeStruct((3, _N), jnp.float32),
        ],
        compiler_params=pltpu.CompilerParams(
            dimension_semantics=("arbitrary",)),
    )(ft, dW1, db1, dW2, db2, cW1, cb1, cW2, cb2)


def kernel(x, emb0, emb1, emb2, emb3, emb4,
           dW1, db1, dW2, db2, cW1, cb1, cW2, cb2):
    parts = []
    for e, cs in zip((emb0, emb1, emb2, emb3, emb4), _COL_STRIDE):
        et = e.T
        hs = et.shape[1]
        if cs != hs:
            et = jnp.pad(et, ((0, 0), (0, cs - hs)))
        parts.append(et.reshape(-1))
    tab = jnp.concatenate(parts)
    x0 = x[:, 0]
    x1 = x[:, 1]
    x2 = x[:, 2]
    ft = _encode(x0, x1, x2, tab)
    dense_r, color_r = _mlp(ft, dW1.T, db1[:, None], dW2, db2[:, None],
                            cW1.T, cb1[:, None], cW2, cb2[:, None])
    return (dense_r.reshape(-1, 1), color_r.T)
